# baseline (speedup 1.0000x reference)
import functools

import jax
import jax.numpy as jnp
from jax import lax
from jax.experimental import pallas as pl
from jax.experimental.pallas import tpu as pltpu
from jax.experimental.pallas import tpu_sc as plsc

NC, NS, L = 2, 16, 16
NW = NC * NS
D = 64
V = 1_000_000
NJ = V // 128
TK = 500_032
B = 4096
T = 200

_PARAMS = pltpu.CompilerParams(
    use_tc_tiling_on_sc=True, needs_layout_passes=False)


def _mesh():
    return plsc.VectorSubcoreMesh(
        core_axis_name="c", subcore_axis_name="s",
        num_cores=NC, num_subcores=NS)


@functools.partial(
    pl.kernel, mesh=_mesh(), compiler_params=_PARAMS,
    out_type=jax.ShapeDtypeStruct((TK, 128), jnp.float32),
    scratch_types=[
        pltpu.VMEM((2, D, 128), jnp.float32),
        pltpu.VMEM((2, D, 128), jnp.float32),
        pltpu.SemaphoreType.DMA,
        pltpu.SemaphoreType.DMA,
    ],
)
def _pack_table(wt_hbm, tail_hbm, tab_hbm, wt_v, tr_v, rsem, wsem):
    c = lax.axis_index("c")
    s = lax.axis_index("s")
    wid = s * NC + c
    iot = lax.iota(jnp.int32, L)

    def rd(j, buf):
        return pltpu.make_async_copy(
            wt_hbm.at[:, pl.ds(j * 128, 128)], wt_v.at[buf], rsem)

    def wr(j, buf):
        return pltpu.make_async_copy(
            tr_v.at[buf], tab_hbm.at[pl.ds(j * (128 // 2), D)], wsem)

    def transpose_block(buf, ncols):
        for cc in range(ncols):
            csp = jnp.full((L,), cc, jnp.int32)
            for ch in range(0, D, L):
                v = plsc.load_gather(wt_v.at[buf], [iot + ch, csp])
                tr_v[buf, cc // 2, pl.ds((cc % 2) * D + ch, L)] = v

    n_i = (NJ - wid + NW - 1) // NW

    def jof(i):
        return wid + i * NW

    rd(jof(0), 0).start()

    @pl.loop(0, n_i, step=2)
    def _(i0):
        for b in range(2):
            i = i0 + b

            @pl.when(i < n_i)
            def _():
                rd(jof(i), b).wait()

                @pl.when(i + 1 < n_i)
                def _():
                    rd(jof(i + 1), 1 - b).start()

                transpose_block(b, 128)

                @pl.when(i > 0)
                def _():
                    wr(jof(i - 1), 1 - b).wait()

                wr(jof(i), b).start()

    wr(jof(n_i - 1), (n_i - 1) % 2).wait()

    @pl.when(wid == NW - 1)
    def _():
        pltpu.sync_copy(tail_hbm, wt_v.at[0])
        transpose_block(0, 128)
        pltpu.sync_copy(tr_v.at[0], tab_hbm.at[pl.ds(NJ * (128 // 2), D)])


@functools.partial(
    pl.kernel, mesh=_mesh(), compiler_params=_PARAMS,
    out_type=jax.ShapeDtypeStruct((T, D, B), jnp.float32),
    scratch_types=[
        pltpu.VMEM((T, 128), jnp.int32),
        pltpu.VMEM((2, 128), jnp.int32),
        pltpu.VMEM((128,), jnp.int32),
        pltpu.VMEM((2, 128, 128), jnp.float32),
        pltpu.VMEM((2, D, 128), jnp.float32),
        pltpu.SemaphoreType.DMA,
        pltpu.SemaphoreType.DMA,
    ],
)
def _gather(xt_hbm, tab_hbm, out_hbm, idx_v, k2_v, boff_v, rows_v, ob_v,
            gsem, osem):
    c = lax.axis_index("c")
    s = lax.axis_index("s")
    wid = s * NC + c
    iot = lax.iota(jnp.int32, L)

    pltpu.sync_copy(xt_hbm.at[:, pl.ds(wid * 128, 128)], idx_v)

    def compute_k2(t, buf):
        for k in range(128 // L):
            v = idx_v[t, pl.ds(k * L, L)]
            k2_v[buf, pl.ds(k * L, L)] = v >> 1

    def g(buf):
        return pltpu.make_async_copy(
            tab_hbm.at[k2_v.at[buf]], rows_v.at[buf], gsem)

    def wr(t, buf):
        return pltpu.make_async_copy(
            ob_v.at[buf], out_hbm.at[t, :, pl.ds(wid * 128, 128)], osem)

    def build_block(t, buf):
        for k in range(128 // L):
            v = idx_v[t, pl.ds(k * L, L)]
            boff_v[pl.ds(k * L, L)] = (v & 1) << 6
        for ch in range(128 // L):
            cvec = iot + ch * L
            cb = boff_v[pl.ds(ch * L, L)]
            for d in range(D):
                vv = plsc.load_gather(rows_v.at[buf], [cvec, cb + d])
                ob_v[buf, d, pl.ds(ch * L, L)] = vv

    compute_k2(0, 0)
    g(0).start()

    @pl.loop(0, T, step=2)
    def _(t0):
        for b in range(2):
            t = t0 + b

            @pl.when(t + 1 < T)
            def _():
                compute_k2(t + 1, 1 - b)
                g(1 - b).start()

            g(b).wait()
            build_block(t, b)

            @pl.when(t > 0)
            def _():
                wr(t - 1, 1 - b).wait()

            wr(t, b).start()

    wr(T - 1, (T - 1) % 2).wait()


def kernel(x, weight):
    xt = jnp.transpose(x).astype(jnp.int32)
    wt = jnp.transpose(weight)
    tail = jnp.pad(lax.slice(wt, (0, NJ * 128), (D, V)), ((0, 0), (0, 64)))
    table = _pack_table(wt, tail)
    out3 = _gather(xt, table)
    return jnp.transpose(out3, (2, 0, 1))


# speedup vs baseline: 1.4794x; 1.4794x over previous
import functools

import jax
import jax.numpy as jnp
from jax import lax
from jax.experimental import pallas as pl
from jax.experimental.pallas import tpu as pltpu
from jax.experimental.pallas import tpu_sc as plsc

NC, NS, L = 2, 16, 16
NW = NC * NS
D = 64
V = 1_000_000
NJ = V // 128
TK = 500_032
B = 4096
T = 200

_PARAMS = pltpu.CompilerParams(
    use_tc_tiling_on_sc=True, needs_layout_passes=False)


def _mesh():
    return plsc.VectorSubcoreMesh(
        core_axis_name="c", subcore_axis_name="s",
        num_cores=NC, num_subcores=NS)


@functools.partial(
    pl.kernel, mesh=_mesh(), compiler_params=_PARAMS,
    out_type=jax.ShapeDtypeStruct((TK, 128), jnp.float32),
    scratch_types=[
        pltpu.VMEM((2, D, 128), jnp.float32),
        pltpu.VMEM((2, D, 128), jnp.float32),
        pltpu.SemaphoreType.DMA,
        pltpu.SemaphoreType.DMA,
    ],
)
def _pack_table(wt_hbm, tail_hbm, tab_hbm, wt_v, tr_v, rsem, wsem):
    c = lax.axis_index("c")
    s = lax.axis_index("s")
    wid = s * NC + c
    iot = lax.iota(jnp.int32, L)

    def rd(j, buf):
        return pltpu.make_async_copy(
            wt_hbm.at[:, pl.ds(j * 128, 128)], wt_v.at[buf], rsem)

    def wr(j, buf):
        return pltpu.make_async_copy(
            tr_v.at[buf], tab_hbm.at[pl.ds(j * (128 // 2), D)], wsem)

    def transpose_block(buf, ncols):
        flat = [(cc, ch) for cc in range(ncols) for ch in range(0, D, L)]
        for g0 in range(0, len(flat), 8):
            grp = flat[g0:g0 + 8]
            vals = [
                plsc.load_gather(
                    wt_v.at[buf], [iot + ch, jnp.full((L,), cc, jnp.int32)])
                for cc, ch in grp
            ]
            for (cc, ch), v in zip(grp, vals):
                tr_v[buf, cc // 2, pl.ds((cc % 2) * D + ch, L)] = v

    n_i = (NJ - wid + NW - 1) // NW

    def jof(i):
        return wid + i * NW

    rd(jof(0), 0).start()

    @pl.loop(0, n_i, step=2)
    def _(i0):
        for b in range(2):
            i = i0 + b

            @pl.when(i < n_i)
            def _():
                rd(jof(i), b).wait()

                @pl.when(i + 1 < n_i)
                def _():
                    rd(jof(i + 1), 1 - b).start()

                transpose_block(b, 128)

                @pl.when(i > 0)
                def _():
                    wr(jof(i - 1), 1 - b).wait()

                wr(jof(i), b).start()

    wr(jof(n_i - 1), (n_i - 1) % 2).wait()

    @pl.when(wid == NW - 1)
    def _():
        pltpu.sync_copy(tail_hbm, wt_v.at[0])
        transpose_block(0, 128)
        pltpu.sync_copy(tr_v.at[0], tab_hbm.at[pl.ds(NJ * (128 // 2), D)])


@functools.partial(
    pl.kernel, mesh=_mesh(), compiler_params=_PARAMS,
    out_type=jax.ShapeDtypeStruct((T, D, B), jnp.float32),
    scratch_types=[
        pltpu.VMEM((T, 128), jnp.int32),
        pltpu.VMEM((2, 128), jnp.int32),
        pltpu.VMEM((128,), jnp.int32),
        pltpu.VMEM((2, 128, 128), jnp.float32),
        pltpu.VMEM((2, D, 128), jnp.float32),
        pltpu.SemaphoreType.DMA,
        pltpu.SemaphoreType.DMA,
    ],
)
def _gather(xt_hbm, tab_hbm, out_hbm, idx_v, k2_v, boff_v, rows_v, ob_v,
            gsem, osem):
    c = lax.axis_index("c")
    s = lax.axis_index("s")
    wid = s * NC + c
    iot = lax.iota(jnp.int32, L)

    pltpu.sync_copy(xt_hbm.at[:, pl.ds(wid * 128, 128)], idx_v)

    def compute_k2(t, buf):
        for k in range(128 // L):
            v = idx_v[t, pl.ds(k * L, L)]
            k2_v[buf, pl.ds(k * L, L)] = v >> 1

    def g(buf):
        return pltpu.make_async_copy(
            tab_hbm.at[k2_v.at[buf]], rows_v.at[buf], gsem)

    def wr(t, buf):
        return pltpu.make_async_copy(
            ob_v.at[buf], out_hbm.at[t, :, pl.ds(wid * 128, 128)], osem)

    def build_block(t, buf):
        for k in range(128 // L):
            v = idx_v[t, pl.ds(k * L, L)]
            boff_v[pl.ds(k * L, L)] = (v & 1) << 6
        for ch in range(128 // L):
            cvec = iot + ch * L
            cb = boff_v[pl.ds(ch * L, L)]
            for d0 in range(0, D, 8):
                vals = [
                    plsc.load_gather(rows_v.at[buf], [cvec, cb + (d0 + g)])
                    for g in range(8)
                ]
                for g in range(8):
                    ob_v[buf, d0 + g, pl.ds(ch * L, L)] = vals[g]

    compute_k2(0, 0)
    g(0).start()

    @pl.loop(0, T, step=2)
    def _(t0):
        for b in range(2):
            t = t0 + b

            @pl.when(t + 1 < T)
            def _():
                compute_k2(t + 1, 1 - b)
                g(1 - b).start()

            g(b).wait()
            build_block(t, b)

            @pl.when(t > 0)
            def _():
                wr(t - 1, 1 - b).wait()

            wr(t, b).start()

    wr(T - 1, (T - 1) % 2).wait()


def kernel(x, weight):
    xt = jnp.transpose(x).astype(jnp.int32)
    wt = jnp.transpose(weight)
    tail = jnp.pad(lax.slice(wt, (0, NJ * 128), (D, V)), ((0, 0), (0, 64)))
    table = _pack_table(wt, tail)
    out3 = _gather(xt, table)
    return jnp.transpose(out3, (2, 0, 1))


# speedup vs baseline: 2.3122x; 1.5630x over previous
import functools

import jax
import jax.numpy as jnp
from jax import lax
from jax.experimental import pallas as pl
from jax.experimental.pallas import tpu as pltpu
from jax.experimental.pallas import tpu_sc as plsc

NC = 2
NS = 16
NW = NC * NS

D = 64
CHUNK = 512


def _make_gather(B):
    assert B % (NW * CHUNK) == 0
    b_per_w = B // NW
    n_super = b_per_w // CHUNK
    mesh = plsc.VectorSubcoreMesh(
        core_axis_name="c", subcore_axis_name="s",
        num_cores=NC, num_subcores=NS)

    @functools.partial(
        pl.kernel,
        mesh=mesh,
        compiler_params=pltpu.CompilerParams(use_tc_tiling_on_sc=False),
        out_type=jax.ShapeDtypeStruct((B, D), jnp.float32),
        scratch_types=[
            pltpu.VMEM((b_per_w,), jnp.int32),
            pltpu.VMEM((2, CHUNK, D), jnp.float32),
            pltpu.SemaphoreType.DMA,
            pltpu.SemaphoreType.DMA,
        ],
    )
    def gather_kernel(idx_hbm, table_hbm, out_hbm, idx_v, rows_v, gsem, osem):
        c = lax.axis_index("c")
        s = lax.axis_index("s")
        wid = s * NC + c
        base = wid * b_per_w

        pltpu.sync_copy(idx_hbm.at[pl.ds(base, b_per_w)], idx_v)

        def gather_descr(sidx, buf):
            return pltpu.make_async_copy(
                table_hbm.at[idx_v.at[pl.ds(sidx * CHUNK, CHUNK)]],
                rows_v.at[buf],
                gsem)

        def out_descr(sidx, buf):
            return pltpu.make_async_copy(
                rows_v.at[buf],
                out_hbm.at[pl.ds(base + sidx * CHUNK, CHUNK)],
                osem)

        gather_descr(0, 0).start()

        @pl.loop(0, n_super, step=2)
        def _(si):
            for b in range(2):
                sidx = si + b
                gather_descr(sidx, b).wait()

                @pl.when(sidx > 0)
                def _():
                    out_descr(sidx - 1, 1 - b).wait()

                @pl.when(sidx + 1 < n_super)
                def _():
                    gather_descr(sidx + 1, 1 - b).start()

                out_descr(sidx, b).start()

        out_descr(n_super - 1, (n_super - 1) % 2).wait()

    return gather_kernel


def kernel(x, weight):
    B = x.size
    idx = x.reshape(-1).astype(jnp.int32)
    out = _make_gather(B)(idx, weight)
    return out.reshape(*x.shape, D)
